# baseline (speedup 1.0000x reference)
import functools

import jax
import jax.numpy as jnp
from jax import lax
from jax.experimental import pallas as pl
from jax.experimental.pallas import tpu as pltpu
from jax.experimental.pallas import tpu_sc as plsc

NC = 2
NS = 16
NW = NC * NS

CHUNK = 4096
NCH = 25


def _lookup_kernel(x_hbm, params_hbm, out_hbm, idx0, idx1, out0,
                   out1, tbl_v, par_v, si0, si1, so0, so1):
    wid = lax.axis_index("s") * NC + lax.axis_index("c")

    pltpu.sync_copy(params_hbm, par_v)

    lane = lax.iota(jnp.int32, 16)

    for g in range(10):
        flat = lane + g * 16
        v = flat >> 3
        o = flat & 7
        acc = plsc.load_gather(par_v, [o + 112])
        for d in range(4):
            e = plsc.load_gather(par_v, [v * 4 + d])
            w = plsc.load_gather(par_v, [o * 4 + d + 80])
            acc = acc + e * w
        for r in range(16):
            tbl_v[pl.ds(r * 161 + g * 16, 16)] = acc

    lane161 = lane * 161

    def in_copy(c, buf, sem):
        return pltpu.make_async_copy(
            x_hbm.at[pl.ds(c * (128 * 1024) + wid * CHUNK, CHUNK)], buf, sem)

    def out_copies(c, buf, sem):
        return [pltpu.make_async_copy(
                    buf.at[pl.ds(j_s * 4096, 4096)],
                    out_hbm.at[pl.ds(
                        c * (8 * 128 * 1024) + j_s * (128 * 1024)
                        + wid * 4096, 4096)],
                    sem)
                for j_s in range(8)]

    def out_start(c, buf, sem):
        for cp in out_copies(c, buf, sem):
            cp.start()

    def out_wait(c, buf, sem):
        for cp in out_copies(c, buf, sem):
            cp.wait()

    def compute(c, idx_v, out_v, sem):
        @plsc.parallel_loop(0, 32, 1, unroll=4)
        def slab(sl):
            ib = sl * 128
            ob = (sl & 7) * 4096 + (sl >> 3) * 1024
            for u in range(8):
                idxs = idx_v[pl.ds(ib + u * 16, 16)]
                ga = idxs * 8 + lane161
                for k in range(8):
                    vals = plsc.load_gather(tbl_v, [ga + k])
                    out_v[pl.ds(ob + k * 128 + u * 16, 16)] = vals

        out_start(c, out_v, sem)

    in_copy(0, idx0, si0).start()

    def pair_body(t, carry):
        c0 = 2 * t
        in_copy(c0 + 1, idx1, si1).start()
        in_copy(c0, idx0, si0).wait()

        @pl.when(t >= 1)
        def _():
            out_wait(c0 - 2, out0, so0)

        compute(c0, idx0, out0, so0)

        in_copy(c0 + 2, idx0, si0).start()
        in_copy(c0 + 1, idx1, si1).wait()

        @pl.when(t >= 1)
        def _():
            out_wait(c0 - 1, out1, so1)

        compute(c0 + 1, idx1, out1, so1)
        return carry

    lax.fori_loop(0, (NCH - 1) // 2, pair_body, 0)

    in_copy(NCH - 1, idx0, si0).wait()
    out_wait(NCH - 3, out0, so0)
    compute(NCH - 1, idx0, out0, so0)

    out_wait(NCH - 2, out1, so1)
    out_wait(NCH - 1, out0, so0)


def kernel(x, embed_table, W, b):
    B, L = x.shape
    n = B * L
    xr = (x.astype(jnp.int32)
          .reshape(B // 128, 128, L // 8, 8)
          .transpose(2, 0, 3, 1)
          .reshape(n))
    params = jnp.concatenate([
        embed_table.reshape(-1).astype(jnp.float32),
        W.reshape(-1).astype(jnp.float32),
        b.astype(jnp.float32),
        jnp.zeros((8,), jnp.float32),
    ])

    mesh = plsc.VectorSubcoreMesh(core_axis_name="c", subcore_axis_name="s")
    out_flat = pl.kernel(
        _lookup_kernel,
        out_type=jax.ShapeDtypeStruct((n * 8,), jnp.float32),
        mesh=mesh,
        scratch_types=[
            pltpu.VMEM((CHUNK,), jnp.int32),
            pltpu.VMEM((CHUNK,), jnp.int32),
            pltpu.VMEM((CHUNK * 8,), jnp.float32),
            pltpu.VMEM((CHUNK * 8,), jnp.float32),
            pltpu.VMEM((16 * 161,), jnp.float32),
            pltpu.VMEM((128,), jnp.float32),
            pltpu.SemaphoreType.DMA,
            pltpu.SemaphoreType.DMA,
            pltpu.SemaphoreType.DMA,
            pltpu.SemaphoreType.DMA,
        ],
        compiler_params=pltpu.CompilerParams(needs_layout_passes=False),
    )(xr, params)
    out = (out_flat.reshape(L, B // 128, 8, 128)
           .transpose(1, 3, 0, 2)
           .reshape(B, L, 8))
    return out


# speedup vs baseline: 1.0229x; 1.0229x over previous
import functools

import jax
import jax.numpy as jnp
from jax import lax
from jax.experimental import pallas as pl
from jax.experimental.pallas import tpu as pltpu
from jax.experimental.pallas import tpu_sc as plsc

NC = 2
NS = 16
NW = NC * NS

CHUNK = 4096
NCH = 25


def _lookup_kernel(x_hbm, params_hbm, out_hbm, idx0, idx1, out0,
                   out1, tbl_v, par_v, si0, si1, so0, so1):
    wid = lax.axis_index("s") * NC + lax.axis_index("c")

    pltpu.sync_copy(params_hbm, par_v)

    lane = lax.iota(jnp.int32, 16)

    for g in range(10):
        flat = lane + g * 16
        v = flat >> 3
        o = flat & 7
        acc = plsc.load_gather(par_v, [o + 112])
        for d in range(4):
            e = plsc.load_gather(par_v, [v * 4 + d])
            w = plsc.load_gather(par_v, [o * 4 + d + 80])
            acc = acc + e * w
        for r in range(16):
            plsc.store_scatter(tbl_v, [r * 321 + v * 16 + o], acc)

    lane321 = lane * 321

    def in_copy(c, buf, sem):
        return pltpu.make_async_copy(
            x_hbm.at[pl.ds(c * (128 * 1024) + wid * CHUNK, CHUNK)], buf, sem)

    def out_copies(c, buf, sem):
        return [pltpu.make_async_copy(
                    buf.at[pl.ds(j_s * 4096, 4096)],
                    out_hbm.at[pl.ds(
                        c * (8 * 128 * 1024) + j_s * (128 * 1024)
                        + wid * 4096, 4096)],
                    sem)
                for j_s in range(8)]

    def out_start(c, buf, sem):
        for cp in out_copies(c, buf, sem):
            cp.start()

    def out_wait(c, buf, sem):
        for cp in out_copies(c, buf, sem):
            cp.wait()

    def compute(c, idx_v, out_v, sem):
        @plsc.parallel_loop(0, 32, 1, unroll=2)
        def slab(sl):
            ib = sl * 128
            ob = (sl & 7) * 4096 + (sl >> 3) * 1024
            for u in range(8):
                idxs = idx_v[pl.ds(ib + u * 16, 16)]
                ga = idxs * 16 + lane321
                for k in range(8):
                    vals = plsc.load_gather(tbl_v, [ga + k])
                    out_v[pl.ds(ob + k * 128 + u * 16, 16)] = vals

        out_start(c, out_v, sem)

    in_copy(0, idx0, si0).start()

    def pair_body(t, carry):
        c0 = 2 * t
        in_copy(c0 + 1, idx1, si1).start()
        in_copy(c0, idx0, si0).wait()

        @pl.when(t >= 1)
        def _():
            out_wait(c0 - 2, out0, so0)

        compute(c0, idx0, out0, so0)

        in_copy(c0 + 2, idx0, si0).start()
        in_copy(c0 + 1, idx1, si1).wait()

        @pl.when(t >= 1)
        def _():
            out_wait(c0 - 1, out1, so1)

        compute(c0 + 1, idx1, out1, so1)
        return carry

    lax.fori_loop(0, (NCH - 1) // 2, pair_body, 0)

    in_copy(NCH - 1, idx0, si0).wait()
    out_wait(NCH - 3, out0, so0)
    compute(NCH - 1, idx0, out0, so0)

    out_wait(NCH - 2, out1, so1)
    out_wait(NCH - 1, out0, so0)


def kernel(x, embed_table, W, b):
    B, L = x.shape
    n = B * L
    xr = (x.astype(jnp.int32)
          .reshape(B // 128, 128, L // 8, 8)
          .transpose(2, 0, 3, 1)
          .reshape(n))
    params = jnp.concatenate([
        embed_table.reshape(-1).astype(jnp.float32),
        W.reshape(-1).astype(jnp.float32),
        b.astype(jnp.float32),
        jnp.zeros((8,), jnp.float32),
    ])

    mesh = plsc.VectorSubcoreMesh(core_axis_name="c", subcore_axis_name="s")
    out_flat = pl.kernel(
        _lookup_kernel,
        out_type=jax.ShapeDtypeStruct((n * 8,), jnp.float32),
        mesh=mesh,
        scratch_types=[
            pltpu.VMEM((CHUNK,), jnp.int32),
            pltpu.VMEM((CHUNK,), jnp.int32),
            pltpu.VMEM((CHUNK * 8,), jnp.float32),
            pltpu.VMEM((CHUNK * 8,), jnp.float32),
            pltpu.VMEM((16 * 321,), jnp.float32),
            pltpu.VMEM((128,), jnp.float32),
            pltpu.SemaphoreType.DMA,
            pltpu.SemaphoreType.DMA,
            pltpu.SemaphoreType.DMA,
            pltpu.SemaphoreType.DMA,
        ],
        compiler_params=pltpu.CompilerParams(needs_layout_passes=False),
    )(xr, params)
    out = (out_flat.reshape(L, B // 128, 8, 128)
           .transpose(1, 3, 0, 2)
           .reshape(B, L, 8))
    return out


# speedup vs baseline: 1.1800x; 1.1535x over previous
import jax
import jax.numpy as jnp
from jax import lax
from jax.experimental import pallas as pl
from jax.experimental.pallas import tpu as pltpu
from jax.experimental.pallas import tpu_sc as plsc

NC = 2
NS = 16
NW = NC * NS

CHUNK = 4096
NCH = 25


def _lookup_kernel(x_hbm, params_hbm, out_hbm, idx0, idx1, out0,
                   out1, tbl_v, par_v, si0, si1, so0, so1):
    wid = lax.axis_index("s") * NC + lax.axis_index("c")

    pltpu.sync_copy(params_hbm, par_v)

    lane = lax.iota(jnp.int32, 16)

    for g in range(10):
        flat = lane + g * 16
        v = flat >> 3
        o = flat & 7
        acc = plsc.load_gather(par_v, [o + 112])
        for d in range(4):
            e = plsc.load_gather(par_v, [v * 4 + d])
            w = plsc.load_gather(par_v, [o * 4 + d + 80])
            acc = acc + e * w
        for r in range(16):
            tbl_v[pl.ds(r * 161 + g * 16, 16)] = acc

    lane161 = lane * 161

    def in_copy(c, buf, sem):
        return pltpu.make_async_copy(
            x_hbm.at[pl.ds(c * (128 * 1024) + wid * CHUNK, CHUNK)], buf, sem)

    def out_copies(c, buf, sem):
        return [pltpu.make_async_copy(
                    buf.at[pl.ds(j_s * 4096, 4096)],
                    out_hbm.at[pl.ds(
                        c * (8 * 128 * 1024) + j_s * (128 * 1024)
                        + wid * 4096, 4096)],
                    sem)
                for j_s in range(8)]

    def out_start(c, buf, sem):
        for cp in out_copies(c, buf, sem):
            cp.start()

    def out_wait(c, buf, sem):
        for cp in out_copies(c, buf, sem):
            cp.wait()

    def compute(c, idx_v, out_v, sem):
        @plsc.parallel_loop(0, 32, 1, unroll=2)
        def slab(sl):
            ib = sl * 128
            ob = (sl & 7) * 4096 + (sl >> 3) * 1024
            for u in range(8):
                idxs = idx_v[pl.ds(ib + u * 16, 16)]
                ga = idxs * 8 + lane161
                for k in range(8):
                    vals = plsc.load_gather(tbl_v, [ga + k])
                    out_v[pl.ds(ob + k * 128 + u * 16, 16)] = vals

        out_start(c, out_v, sem)

    in_copy(0, idx0, si0).start()

    def pair_body(t, carry):
        c0 = 2 * t
        in_copy(c0 + 1, idx1, si1).start()
        in_copy(c0, idx0, si0).wait()

        @pl.when(t >= 1)
        def _():
            out_wait(c0 - 2, out0, so0)

        compute(c0, idx0, out0, so0)

        in_copy(c0 + 2, idx0, si0).start()
        in_copy(c0 + 1, idx1, si1).wait()

        @pl.when(t >= 1)
        def _():
            out_wait(c0 - 1, out1, so1)

        compute(c0 + 1, idx1, out1, so1)
        return carry

    lax.fori_loop(0, (NCH - 1) // 2, pair_body, 0)

    in_copy(NCH - 1, idx0, si0).wait()
    out_wait(NCH - 3, out0, so0)
    compute(NCH - 1, idx0, out0, so0)

    out_wait(NCH - 2, out1, so1)
    out_wait(NCH - 1, out0, so0)


def kernel(x, embed_table, W, b):
    B, L = x.shape
    n = B * L
    xr = (x.astype(jnp.int32)
          .reshape(B // 128, 128, L // 8, 8)
          .transpose(2, 0, 3, 1)
          .reshape(n))
    params = jnp.concatenate([
        embed_table.reshape(-1).astype(jnp.float32),
        W.reshape(-1).astype(jnp.float32),
        b.astype(jnp.float32),
        jnp.zeros((8,), jnp.float32),
    ])

    mesh = plsc.VectorSubcoreMesh(core_axis_name="c", subcore_axis_name="s")
    out_flat = pl.kernel(
        _lookup_kernel,
        out_type=jax.ShapeDtypeStruct((n * 8,), jnp.float32),
        mesh=mesh,
        scratch_types=[
            pltpu.VMEM((CHUNK,), jnp.int32),
            pltpu.VMEM((CHUNK,), jnp.int32),
            pltpu.VMEM((CHUNK * 8,), jnp.float32),
            pltpu.VMEM((CHUNK * 8,), jnp.float32),
            pltpu.VMEM((16 * 161,), jnp.float32),
            pltpu.VMEM((128,), jnp.float32),
            pltpu.SemaphoreType.DMA,
            pltpu.SemaphoreType.DMA,
            pltpu.SemaphoreType.DMA,
            pltpu.SemaphoreType.DMA,
        ],
        compiler_params=pltpu.CompilerParams(needs_layout_passes=False),
    )(xr, params)
    out = (out_flat.reshape(L, B // 128, 8, 128)
           .transpose(1, 3, 0, 2)
           .reshape(B, L, 8))
    return out


# speedup vs baseline: 1.2031x; 1.0196x over previous
import jax
import jax.numpy as jnp
from jax import lax
from jax.experimental import pallas as pl
from jax.experimental.pallas import tpu as pltpu
from jax.experimental.pallas import tpu_sc as plsc

NC = 2
NS = 16
NW = NC * NS

CHUNK = 4096
NCH = 25


def _lookup_kernel(x_hbm, params_hbm, out_hbm, idx0, idx1, out0,
                   out1, tbl_v, par_v, si0, si1, so0, so1):
    wid = lax.axis_index("s") * NC + lax.axis_index("c")

    pltpu.sync_copy(params_hbm, par_v)

    lane = lax.iota(jnp.int32, 16)

    for g in range(10):
        flat = lane + g * 16
        v = flat >> 3
        o = flat & 7
        acc = plsc.load_gather(par_v, [o + 112])
        for d in range(4):
            e = plsc.load_gather(par_v, [v * 4 + d])
            w = plsc.load_gather(par_v, [o * 4 + d + 80])
            acc = acc + e * w
        for r in range(8):
            tbl_v[pl.ds(r * 161 + g * 16, 16)] = acc

    lane161 = (lane & 7) * 161

    def in_copy(c, buf, sem):
        return pltpu.make_async_copy(
            x_hbm.at[pl.ds(c * (128 * 1024) + wid * CHUNK, CHUNK)], buf, sem)

    def out_copies(c, buf, sem):
        return [pltpu.make_async_copy(
                    buf.at[pl.ds(j_s * 4096, 4096)],
                    out_hbm.at[pl.ds(
                        c * (8 * 128 * 1024) + j_s * (128 * 1024)
                        + wid * 4096, 4096)],
                    sem)
                for j_s in range(8)]

    def out_start(c, buf, sem):
        for cp in out_copies(c, buf, sem):
            cp.start()

    def out_wait(c, buf, sem):
        for cp in out_copies(c, buf, sem):
            cp.wait()

    def compute(c, idx_v, out_v, sem):
        @plsc.parallel_loop(0, 32, 1, unroll=2)
        def slab(sl):
            ib = sl * 128
            ob = (sl & 7) * 4096 + (sl >> 3) * 1024
            for u in range(8):
                idxs = idx_v[pl.ds(ib + u * 16, 16)]
                ga = idxs * 8 + lane161
                for k in range(8):
                    vals = plsc.load_gather(tbl_v, [ga + k])
                    out_v[pl.ds(ob + k * 128 + u * 16, 16)] = vals

        out_start(c, out_v, sem)

    in_copy(0, idx0, si0).start()

    def pair_body(t, carry):
        c0 = 2 * t
        in_copy(c0 + 1, idx1, si1).start()
        in_copy(c0, idx0, si0).wait()

        @pl.when(t >= 1)
        def _():
            out_wait(c0 - 2, out0, so0)

        compute(c0, idx0, out0, so0)

        in_copy(c0 + 2, idx0, si0).start()
        in_copy(c0 + 1, idx1, si1).wait()

        @pl.when(t >= 1)
        def _():
            out_wait(c0 - 1, out1, so1)

        compute(c0 + 1, idx1, out1, so1)
        return carry

    lax.fori_loop(0, (NCH - 1) // 2, pair_body, 0)

    in_copy(NCH - 1, idx0, si0).wait()
    out_wait(NCH - 3, out0, so0)
    compute(NCH - 1, idx0, out0, so0)

    out_wait(NCH - 2, out1, so1)
    out_wait(NCH - 1, out0, so0)


def kernel(x, embed_table, W, b):
    B, L = x.shape
    n = B * L
    xr = (x.astype(jnp.int32)
          .reshape(B // 128, 128, L // 8, 8)
          .transpose(2, 0, 3, 1)
          .reshape(n))
    params = jnp.concatenate([
        embed_table.reshape(-1).astype(jnp.float32),
        W.reshape(-1).astype(jnp.float32),
        b.astype(jnp.float32),
        jnp.zeros((8,), jnp.float32),
    ])

    mesh = plsc.VectorSubcoreMesh(core_axis_name="c", subcore_axis_name="s")
    out_flat = pl.kernel(
        _lookup_kernel,
        out_type=jax.ShapeDtypeStruct((n * 8,), jnp.float32),
        mesh=mesh,
        scratch_types=[
            pltpu.VMEM((CHUNK,), jnp.int32),
            pltpu.VMEM((CHUNK,), jnp.int32),
            pltpu.VMEM((CHUNK * 8,), jnp.float32),
            pltpu.VMEM((CHUNK * 8,), jnp.float32),
            pltpu.VMEM((8 * 161,), jnp.float32),
            pltpu.VMEM((128,), jnp.float32),
            pltpu.SemaphoreType.DMA,
            pltpu.SemaphoreType.DMA,
            pltpu.SemaphoreType.DMA,
            pltpu.SemaphoreType.DMA,
        ],
        compiler_params=pltpu.CompilerParams(needs_layout_passes=False),
    )(xr, params)
    out = (out_flat.reshape(L, B // 128, 8, 128)
           .transpose(1, 3, 0, 2)
           .reshape(B, L, 8))
    return out
